# baseline (speedup 1.0000x reference)
import functools

import numpy as np
import jax
import jax.numpy as jnp
from jax.experimental import pallas as pl
from jax.experimental.pallas import tpu as pltpu

B, S, DIM = 2, 2048, 2048
Q_LORA = 1536
H, D = 16, 128
R = 64
HALF = R // 2

import ml_dtypes
_SCALE = float(np.array(D ** -0.5, dtype=ml_dtypes.bfloat16))


def _hadamard_matrix(d: int) -> np.ndarray:
    x = np.eye(d, dtype=np.float64)
    h = 1
    while h < d:
        x = x.reshape(-1, d // (2 * h), 2, h)
        a = x[..., 0, :]
        b = x[..., 1, :]
        x = np.stack([a + b, a - b], axis=-2)
        h *= 2
    return x.reshape(d, d)


_HAD = _hadamard_matrix(D).astype(ml_dtypes.bfloat16)


def _rope_had(v, cos, sa, sb, had):
    v = v * cos + pltpu.roll(v, D - HALF, axis=1) * sa + pltpu.roll(v, HALF, axis=1) * sb
    vb = v.astype(jnp.bfloat16)
    return (jnp.dot(vb, had, preferred_element_type=jnp.float32)
            * _SCALE).astype(jnp.bfloat16)


def _body(ql_ref, x_ref, wq_ref, wk_ref, bq_ref, bk_ref, g_ref, beta_ref,
          cos_ref, sa_ref, sb_ref, had_ref, qo_ref, ko_ref):
    cos = cos_ref[...]
    sa = sa_ref[...]
    sb = sb_ref[...]
    had = had_ref[...]

    k = jax.lax.dot_general(x_ref[0].astype(jnp.bfloat16), wk_ref[...],
                            (((1,), (1,)), ((), ())),
                            preferred_element_type=jnp.float32)
    k = k + bk_ref[...]
    mu = jnp.mean(k, axis=-1, keepdims=True)
    var = jnp.mean((k - mu) ** 2, axis=-1, keepdims=True)
    k = (k - mu) / jnp.sqrt(var + 1e-5) * g_ref[...] + beta_ref[...]
    ko_ref[0] = _rope_had(k, cos, sa, sb, had)

    q2 = jax.lax.dot_general(ql_ref[0].astype(jnp.bfloat16), wq_ref[...],
                             (((1,), (1,)), ((), ())),
                             preferred_element_type=jnp.float32)
    q2 = q2 + bq_ref[...]
    for h in range(H):
        sl = slice(h * D, (h + 1) * D)
        qo_ref[0, :, sl] = _rope_had(q2[:, sl], cos, sa, sb, had)


@functools.partial(jax.jit, static_argnames=())
def kernel(x, Q_latent, freqs_cis, mask, Wq_up_w, Wq_up_b, Wk_w, Wk_b, ln_g, ln_b):
    del mask
    T = 256

    cos32 = freqs_cis[..., 0]
    sin32 = freqs_cis[..., 1]
    ones = jnp.ones((S, D - R), dtype=jnp.float32)
    zeros = jnp.zeros((S, D - R), dtype=jnp.float32)
    z32 = jnp.zeros((S, HALF), dtype=jnp.float32)
    cos_row = jnp.concatenate([cos32, cos32, ones], axis=1)
    sa_row = jnp.concatenate([-sin32, z32, zeros], axis=1)
    sb_row = jnp.concatenate([z32, sin32, zeros], axis=1)

    wq_bf = Wq_up_w.astype(jnp.bfloat16)
    wk_bf = Wk_w.astype(jnp.bfloat16)
    bq = Wq_up_b.reshape(1, H * D)
    bk = Wk_b.reshape(1, D)
    g = ln_g.reshape(1, D)
    beta = ln_b.reshape(1, D)

    grid = (B, S // T)
    qout, kout = pl.pallas_call(
        _body,
        grid=grid,
        in_specs=[
            pl.BlockSpec((1, T, Q_LORA), lambda b, i: (b, i, 0)),
            pl.BlockSpec((1, T, DIM), lambda b, i: (b, i, 0)),
            pl.BlockSpec((H * D, Q_LORA), lambda b, i: (0, 0)),
            pl.BlockSpec((D, DIM), lambda b, i: (0, 0)),
            pl.BlockSpec((1, H * D), lambda b, i: (0, 0)),
            pl.BlockSpec((1, D), lambda b, i: (0, 0)),
            pl.BlockSpec((1, D), lambda b, i: (0, 0)),
            pl.BlockSpec((1, D), lambda b, i: (0, 0)),
            pl.BlockSpec((T, D), lambda b, i: (i, 0)),
            pl.BlockSpec((T, D), lambda b, i: (i, 0)),
            pl.BlockSpec((T, D), lambda b, i: (i, 0)),
            pl.BlockSpec((D, D), lambda b, i: (0, 0)),
        ],
        out_specs=[
            pl.BlockSpec((1, T, H * D), lambda b, i: (b, i, 0)),
            pl.BlockSpec((1, T, D), lambda b, i: (b, i, 0)),
        ],
        out_shape=[
            jax.ShapeDtypeStruct((B, S, H * D), jnp.bfloat16),
            jax.ShapeDtypeStruct((B, S, D), jnp.bfloat16),
        ],
        compiler_params=pltpu.CompilerParams(
            dimension_semantics=("parallel", "parallel"),
        ),
    )(Q_latent, x, wq_bf, wk_bf, bq, bk, g, beta, cos_row, sa_row, sb_row,
      jnp.asarray(_HAD))

    return qout.reshape(B, S, H, D), kout


# speedup vs baseline: 1.0701x; 1.0701x over previous
import functools

import numpy as np
import jax
import jax.numpy as jnp
from jax.experimental import pallas as pl
from jax.experimental.pallas import tpu as pltpu

B, S, DIM = 2, 2048, 2048
Q_LORA = 1536
H, D = 16, 128
R = 64
HALF = R // 2

import ml_dtypes
_SCALE = float(np.array(D ** -0.5, dtype=ml_dtypes.bfloat16))


def _hadamard_matrix(d: int) -> np.ndarray:
    x = np.eye(d, dtype=np.float64)
    h = 1
    while h < d:
        x = x.reshape(-1, d // (2 * h), 2, h)
        a = x[..., 0, :]
        b = x[..., 1, :]
        x = np.stack([a + b, a - b], axis=-2)
        h *= 2
    return x.reshape(d, d)


_HAD = _hadamard_matrix(D).astype(ml_dtypes.bfloat16)


def _rope_had(v, cos, sa, sb, had):
    v = v * cos + pltpu.roll(v, D - HALF, axis=1) * sa + pltpu.roll(v, HALF, axis=1) * sb
    vb = v.astype(jnp.bfloat16)
    return (jnp.dot(vb, had, preferred_element_type=jnp.float32)
            * _SCALE).astype(jnp.bfloat16)


def _body(ql_ref, x_ref, wq_ref, wk_ref, bq_ref, bk_ref, g_ref, beta_ref,
          cos_ref, sa_ref, sb_ref, had_ref, qo_ref, ko_ref,
          wqT_s, wkT_s):
    cos = cos_ref[...]
    sa = sa_ref[...]
    sb = sb_ref[...]
    had = had_ref[...]

    @pl.when(jnp.logical_and(pl.program_id(0) == 0, pl.program_id(1) == 0))
    def _():
        wqT_s[...] = jnp.transpose(wq_ref[...])
        wkT_s[...] = jnp.transpose(wk_ref[...])

    k = jnp.dot(x_ref[0].astype(jnp.bfloat16), wkT_s[...],
                preferred_element_type=jnp.float32)
    k = k + bk_ref[...]
    mu = jnp.mean(k, axis=-1, keepdims=True)
    var = jnp.mean((k - mu) ** 2, axis=-1, keepdims=True)
    k = (k - mu) / jnp.sqrt(var + 1e-5) * g_ref[...] + beta_ref[...]
    ko_ref[0] = _rope_had(k, cos, sa, sb, had)

    q2 = jnp.dot(ql_ref[0].astype(jnp.bfloat16), wqT_s[...],
                 preferred_element_type=jnp.float32)
    q2 = q2 + bq_ref[...]
    for h in range(H):
        sl = slice(h * D, (h + 1) * D)
        qo_ref[0, :, sl] = _rope_had(q2[:, sl], cos, sa, sb, had)


@functools.partial(jax.jit, static_argnames=())
def kernel(x, Q_latent, freqs_cis, mask, Wq_up_w, Wq_up_b, Wk_w, Wk_b, ln_g, ln_b):
    del mask
    T = 256

    cos32 = freqs_cis[..., 0]
    sin32 = freqs_cis[..., 1]
    ones = jnp.ones((S, D - R), dtype=jnp.float32)
    zeros = jnp.zeros((S, D - R), dtype=jnp.float32)
    z32 = jnp.zeros((S, HALF), dtype=jnp.float32)
    cos_row = jnp.concatenate([cos32, cos32, ones], axis=1)
    sa_row = jnp.concatenate([-sin32, z32, zeros], axis=1)
    sb_row = jnp.concatenate([z32, sin32, zeros], axis=1)

    wq_bf = Wq_up_w.astype(jnp.bfloat16)
    wk_bf = Wk_w.astype(jnp.bfloat16)
    bq = Wq_up_b.reshape(1, H * D)
    bk = Wk_b.reshape(1, D)
    g = ln_g.reshape(1, D)
    beta = ln_b.reshape(1, D)

    grid = (B, S // T)
    qout, kout = pl.pallas_call(
        _body,
        grid=grid,
        in_specs=[
            pl.BlockSpec((1, T, Q_LORA), lambda b, i: (b, i, 0)),
            pl.BlockSpec((1, T, DIM), lambda b, i: (b, i, 0)),
            pl.BlockSpec((H * D, Q_LORA), lambda b, i: (0, 0)),
            pl.BlockSpec((D, DIM), lambda b, i: (0, 0)),
            pl.BlockSpec((1, H * D), lambda b, i: (0, 0)),
            pl.BlockSpec((1, D), lambda b, i: (0, 0)),
            pl.BlockSpec((1, D), lambda b, i: (0, 0)),
            pl.BlockSpec((1, D), lambda b, i: (0, 0)),
            pl.BlockSpec((T, D), lambda b, i: (i, 0)),
            pl.BlockSpec((T, D), lambda b, i: (i, 0)),
            pl.BlockSpec((T, D), lambda b, i: (i, 0)),
            pl.BlockSpec((D, D), lambda b, i: (0, 0)),
        ],
        out_specs=[
            pl.BlockSpec((1, T, H * D), lambda b, i: (b, i, 0)),
            pl.BlockSpec((1, T, D), lambda b, i: (b, i, 0)),
        ],
        out_shape=[
            jax.ShapeDtypeStruct((B, S, H * D), jnp.bfloat16),
            jax.ShapeDtypeStruct((B, S, D), jnp.bfloat16),
        ],
        scratch_shapes=[
            pltpu.VMEM((Q_LORA, H * D), jnp.bfloat16),
            pltpu.VMEM((DIM, D), jnp.bfloat16),
        ],
        compiler_params=pltpu.CompilerParams(
            dimension_semantics=("arbitrary", "arbitrary"),
        ),
    )(Q_latent, x, wq_bf, wk_bf, bq, bk, g, beta, cos_row, sa_row, sb_row,
      jnp.asarray(_HAD))

    return qout.reshape(B, S, H, D), kout


# speedup vs baseline: 1.2125x; 1.1330x over previous
import numpy as np
import ml_dtypes
import jax
import jax.numpy as jnp
from jax.experimental import pallas as pl
from jax.experimental.pallas import tpu as pltpu

B, S, DIM = 2, 2048, 2048
Q_LORA = 1536
H, D = 16, 128
R = 64
HALF = R // 2

_SCALE = float(np.array(D ** -0.5, dtype=ml_dtypes.bfloat16))


def _hadamard_matrix(d: int) -> np.ndarray:
    x = np.eye(d, dtype=np.float64)
    h = 1
    while h < d:
        x = x.reshape(-1, d // (2 * h), 2, h)
        a = x[..., 0, :]
        b = x[..., 1, :]
        x = np.stack([a + b, a - b], axis=-2)
        h *= 2
    return x.reshape(d, d)


_HAD = _hadamard_matrix(D).astype(ml_dtypes.bfloat16)


def _rope_gather_matrix() -> np.ndarray:
    g = np.zeros((R, 3 * D), dtype=np.float32)
    for j in range(HALF):
        g[2 * j, j] = 1.0
        g[2 * j, HALF + j] = 1.0
        g[2 * j + 1, D + j] = -1.0
        g[2 * j + 1, 2 * D + HALF + j] = 1.0
    return g


_ROPE_G = _rope_gather_matrix()


def _rope_had(v, cos, sa, sb, had):
    v = v * cos + pltpu.roll(v, D - HALF, axis=1) * sa + pltpu.roll(v, HALF, axis=1) * sb
    vb = v.astype(jnp.bfloat16)
    return (jnp.dot(vb, had, preferred_element_type=jnp.float32)
            * _SCALE).astype(jnp.bfloat16)


def _body(ql_ref, x_ref, fr_ref, wq_ref, wk_ref, bq_ref, bk_ref, g_ref,
          beta_ref, ropeg_ref, had_ref, qo_ref, ko_ref, wqT_s, wkT_s):
    had = had_ref[...]

    @pl.when(jnp.logical_and(pl.program_id(0) == 0, pl.program_id(1) == 0))
    def _():
        wqT_s[...] = jnp.transpose(wq_ref[...].astype(jnp.bfloat16))
        wkT_s[...] = jnp.transpose(wk_ref[...].astype(jnp.bfloat16))

    t = ql_ref.shape[1]
    rows = jnp.dot(fr_ref[...], ropeg_ref[...], preferred_element_type=jnp.float32)
    lane = jax.lax.broadcasted_iota(jnp.int32, (t, D), 1)
    cos = rows[:, :D] + (lane >= R).astype(jnp.float32)
    sa = rows[:, D:2 * D]
    sb = rows[:, 2 * D:]

    k = jnp.dot(x_ref[0].astype(jnp.bfloat16), wkT_s[...],
                preferred_element_type=jnp.float32)
    k = k + bk_ref[...]
    mu = jnp.mean(k, axis=-1, keepdims=True)
    var = jnp.mean((k - mu) ** 2, axis=-1, keepdims=True)
    k = (k - mu) / jnp.sqrt(var + 1e-5) * g_ref[...] + beta_ref[...]
    ko_ref[0] = _rope_had(k, cos, sa, sb, had)

    q2 = jnp.dot(ql_ref[0].astype(jnp.bfloat16), wqT_s[...],
                 preferred_element_type=jnp.float32)
    q2 = q2 + bq_ref[...]
    for h in range(H):
        sl = slice(h * D, (h + 1) * D)
        qo_ref[0, :, sl] = _rope_had(q2[:, sl], cos, sa, sb, had)


def kernel(x, Q_latent, freqs_cis, mask, Wq_up_w, Wq_up_b, Wk_w, Wk_b, ln_g, ln_b):
    del mask
    T = 256

    fr = freqs_cis.reshape(S, R)
    bq = Wq_up_b.reshape(1, H * D)
    bk = Wk_b.reshape(1, D)
    g = ln_g.reshape(1, D)
    beta = ln_b.reshape(1, D)

    grid = (B, S // T)
    qout, kout = pl.pallas_call(
        _body,
        grid=grid,
        in_specs=[
            pl.BlockSpec((1, T, Q_LORA), lambda b, i: (b, i, 0)),
            pl.BlockSpec((1, T, DIM), lambda b, i: (b, i, 0)),
            pl.BlockSpec((T, R), lambda b, i: (i, 0)),
            pl.BlockSpec((H * D, Q_LORA), lambda b, i: (0, 0)),
            pl.BlockSpec((D, DIM), lambda b, i: (0, 0)),
            pl.BlockSpec((1, H * D), lambda b, i: (0, 0)),
            pl.BlockSpec((1, D), lambda b, i: (0, 0)),
            pl.BlockSpec((1, D), lambda b, i: (0, 0)),
            pl.BlockSpec((1, D), lambda b, i: (0, 0)),
            pl.BlockSpec((R, 3 * D), lambda b, i: (0, 0)),
            pl.BlockSpec((D, D), lambda b, i: (0, 0)),
        ],
        out_specs=[
            pl.BlockSpec((1, T, H * D), lambda b, i: (b, i, 0)),
            pl.BlockSpec((1, T, D), lambda b, i: (b, i, 0)),
        ],
        out_shape=[
            jax.ShapeDtypeStruct((B, S, H * D), jnp.bfloat16),
            jax.ShapeDtypeStruct((B, S, D), jnp.bfloat16),
        ],
        scratch_shapes=[
            pltpu.VMEM((Q_LORA, H * D), jnp.bfloat16),
            pltpu.VMEM((DIM, D), jnp.bfloat16),
        ],
        compiler_params=pltpu.CompilerParams(
            dimension_semantics=("arbitrary", "arbitrary"),
        ),
    )(Q_latent, x, fr, Wq_up_w, Wk_w, bq, bk, g, beta,
      jnp.asarray(_ROPE_G), jnp.asarray(_HAD))

    return qout.reshape(B, S, H, D), kout


# speedup vs baseline: 1.7761x; 1.4649x over previous
import numpy as np
import ml_dtypes
import jax
import jax.numpy as jnp
from jax.experimental import pallas as pl
from jax.experimental.pallas import tpu as pltpu

B, S, DIM = 2, 2048, 2048
Q_LORA = 1536
H, D = 16, 128
R = 64
HALF = R // 2

_SCALE = float(np.array(D ** -0.5, dtype=ml_dtypes.bfloat16))


def _hadamard_matrix(d: int) -> np.ndarray:
    x = np.eye(d, dtype=np.float64)
    h = 1
    while h < d:
        x = x.reshape(-1, d // (2 * h), 2, h)
        a = x[..., 0, :]
        b = x[..., 1, :]
        x = np.stack([a + b, a - b], axis=-2)
        h *= 2
    return x.reshape(d, d)


_HAD = _hadamard_matrix(D).astype(ml_dtypes.bfloat16)


def _rope_gather_matrix() -> np.ndarray:
    g = np.zeros((R, 3 * D), dtype=np.float32)
    for j in range(HALF):
        g[2 * j, j] = 1.0
        g[2 * j, HALF + j] = 1.0
        g[2 * j + 1, D + j] = -1.0
        g[2 * j + 1, 2 * D + HALF + j] = 1.0
    return g


_ROPE_G = _rope_gather_matrix()


def _rope_had(v, cos, sa, sb, had):
    v = v * cos + pltpu.roll(v, D - HALF, axis=1) * sa + pltpu.roll(v, HALF, axis=1) * sb
    vb = v.astype(jnp.bfloat16)
    return (jnp.dot(vb, had, preferred_element_type=jnp.float32)
            * _SCALE).astype(jnp.bfloat16)


def _body(ql_ref, x_ref, fr_ref, wq_ref, wk_ref, bq_ref, bk_ref, g_ref,
          beta_ref, ropeg_ref, had_ref, qo_ref, ko_ref, wqT_s, wkT_s):
    had = had_ref[...]

    @pl.when(jnp.logical_and(pl.program_id(0) == 0, pl.program_id(1) == 0))
    def _():
        wqT_s[...] = jnp.transpose(wq_ref[...].astype(jnp.bfloat16))
        wkT_s[...] = jnp.transpose(wk_ref[...].astype(jnp.bfloat16))

    t = ql_ref.shape[1]
    rows = jnp.dot(fr_ref[...], ropeg_ref[...], preferred_element_type=jnp.float32)
    lane = jax.lax.broadcasted_iota(jnp.int32, (t, D), 1)
    cos = rows[:, :D] + (lane >= R).astype(jnp.float32)
    sa = rows[:, D:2 * D]
    sb = rows[:, 2 * D:]

    k = jnp.dot(x_ref[0].astype(jnp.bfloat16), wkT_s[...],
                preferred_element_type=jnp.float32)
    k = k + bk_ref[...]
    mu = jnp.mean(k, axis=-1, keepdims=True)
    var = jnp.mean((k - mu) ** 2, axis=-1, keepdims=True)
    k = (k - mu) / jnp.sqrt(var + 1e-5) * g_ref[...] + beta_ref[...]
    ko_ref[0] = _rope_had(k, cos, sa, sb, had)

    q2 = jnp.dot(ql_ref[0].astype(jnp.bfloat16), wqT_s[...],
                 preferred_element_type=jnp.float32)
    q2 = q2 + bq_ref[...]
    heads = [_rope_had(q2[:, h * D:(h + 1) * D], cos, sa, sb, had)
             for h in range(H)]
    qcat = jnp.concatenate(heads, axis=1)
    qo_ref[0] = qcat.reshape(qcat.shape[0], H, D)


def kernel(x, Q_latent, freqs_cis, mask, Wq_up_w, Wq_up_b, Wk_w, Wk_b, ln_g, ln_b):
    del mask
    T = 256

    fr = freqs_cis.reshape(S, R)
    bq = Wq_up_b.reshape(1, H * D)
    bk = Wk_b.reshape(1, D)
    g = ln_g.reshape(1, D)
    beta = ln_b.reshape(1, D)

    grid = (B, S // T)
    qout, kout = pl.pallas_call(
        _body,
        grid=grid,
        in_specs=[
            pl.BlockSpec((1, T, Q_LORA), lambda b, i: (b, i, 0)),
            pl.BlockSpec((1, T, DIM), lambda b, i: (b, i, 0)),
            pl.BlockSpec((T, R), lambda b, i: (i, 0)),
            pl.BlockSpec((H * D, Q_LORA), lambda b, i: (0, 0)),
            pl.BlockSpec((D, DIM), lambda b, i: (0, 0)),
            pl.BlockSpec((1, H * D), lambda b, i: (0, 0)),
            pl.BlockSpec((1, D), lambda b, i: (0, 0)),
            pl.BlockSpec((1, D), lambda b, i: (0, 0)),
            pl.BlockSpec((1, D), lambda b, i: (0, 0)),
            pl.BlockSpec((R, 3 * D), lambda b, i: (0, 0)),
            pl.BlockSpec((D, D), lambda b, i: (0, 0)),
        ],
        out_specs=[
            pl.BlockSpec((1, T, H, D), lambda b, i: (b, i, 0, 0)),
            pl.BlockSpec((1, T, D), lambda b, i: (b, i, 0)),
        ],
        out_shape=[
            jax.ShapeDtypeStruct((B, S, H, D), jnp.bfloat16),
            jax.ShapeDtypeStruct((B, S, D), jnp.bfloat16),
        ],
        scratch_shapes=[
            pltpu.VMEM((Q_LORA, H * D), jnp.bfloat16),
            pltpu.VMEM((DIM, D), jnp.bfloat16),
        ],
        compiler_params=pltpu.CompilerParams(
            dimension_semantics=("arbitrary", "arbitrary"),
        ),
    )(Q_latent, x, fr, Wq_up_w, Wk_w, bq, bk, g, beta,
      jnp.asarray(_ROPE_G), jnp.asarray(_HAD))

    return qout, kout
